# baseline (speedup 1.0000x reference)
import jax
import jax.numpy as jnp
import numpy as np
from jax import lax
from jax.experimental import pallas as pl
from jax.experimental.pallas import tpu as pltpu
from jax.experimental.pallas import tpu_sc as plsc

N = 10000
E = 320000
D_IN = 128
D_OUT = 64
D_HID = 128
DF = 64

NC = 2
NS = 16
NW = NC * NS
CH = 128
K32 = 81
K16 = 2 * K32
E_PAD = NW * K32 * CH

PAD_ROWS = 112
N_ACC = N + PAD_ROWS
RPT = N_ACC // NS
PN = 10240
DPT = PN // NS

RB = 1000
GRID = N // RB


def _mesh():
    return plsc.VectorSubcoreMesh(core_axis_name="c", subcore_axis_name="s")


_eps_cache = None


def _eps_values():
    global _eps_cache
    if _eps_cache is None:
        try:
            with jax.default_device(jax.devices("cpu")[0]):
                _eps_cache = np.asarray(
                    jax.random.normal(jax.random.key(42), (N, D_OUT),
                                      jnp.float32))
        except Exception:
            return None
    return _eps_cache


def _deg_body(dst_hbm, zeros_hbm, out_hbm, didx, ones_v, acc,
              sem0, sem1, sem2):
    c = lax.axis_index("c")
    s = lax.axis_index("s")
    wid = s * NC + c
    sems = (sem0, sem1, sem2)
    pltpu.sync_copy(dst_hbm.at[wid], didx)
    pltpu.sync_copy(zeros_hbm.at[pl.ds(s * DPT, DPT)], acc.at[pl.ds(s * DPT, DPT)])
    for i in range(CH // 16):
        ones_v[pl.ds(i * 16, 16)] = jnp.ones((16,), jnp.float32)
    plsc.subcore_barrier()

    for b in range(3):
        pltpu.async_copy(ones_v, acc.at[didx.at[b]], sems[b], add=True)

    def body(jj, carry):
        for b in range(3):
            j = 3 * jj + b
            pltpu.make_async_copy(ones_v, acc.at[didx.at[j]], sems[b]).wait()
            pltpu.async_copy(ones_v, acc.at[didx.at[j + 3]], sems[b], add=True)
        return carry

    lax.fori_loop(0, K32 // 3 - 1, body, 0)
    for b in range(3):
        pltpu.make_async_copy(ones_v, acc.at[didx.at[b]], sems[b]).wait()
    plsc.subcore_barrier()
    pltpu.sync_copy(acc.at[pl.ds(s * DPT, DPT)], out_hbm.at[c, pl.ds(s * DPT, DPT)])


def _sc_deg(dstp, zeros_n):
    return pl.kernel(
        _deg_body,
        out_type=jax.ShapeDtypeStruct((NC, PN), jnp.float32),
        mesh=_mesh(),
        scratch_types=[
            pltpu.VMEM((K32, CH), jnp.int32),
            pltpu.VMEM((CH,), jnp.float32),
            pltpu.VMEM_SHARED((PN,), jnp.float32),
            pltpu.SemaphoreType.DMA,
            pltpu.SemaphoreType.DMA,
            pltpu.SemaphoreType.DMA,
        ],
    )(dstp, zeros_n)


def _agg_body(g_hbm, src_hbm, dst_hbm, zeros_hbm, out_hbm, sidx, didx, rows,
              acc, gsem0, gsem1, gsem2, gsem3, ssem0, ssem1, ssem2, ssem3):
    c = lax.axis_index("c")
    s = lax.axis_index("s")
    gsems = (gsem0, gsem1, gsem2, gsem3)
    ssems = (ssem0, ssem1, ssem2, ssem3)
    pltpu.sync_copy(src_hbm.at[s], sidx)
    pltpu.sync_copy(dst_hbm.at[s], didx)
    pltpu.sync_copy(zeros_hbm.at[pl.ds(s * RPT, RPT)], acc.at[pl.ds(s * RPT, RPT)])
    plsc.subcore_barrier()

    gc = g_hbm.at[c]

    def gather(j, b):
        pltpu.async_copy(gc.at[sidx.at[j]], rows.at[b], gsems[b])

    def gather_wait(j, b):
        pltpu.make_async_copy(gc.at[sidx.at[j]], rows.at[b], gsems[b]).wait()

    def scatter(j, b):
        pltpu.async_copy(rows.at[b], acc.at[didx.at[j]], ssems[b], add=True)

    def scatter_wait(j, b):
        pltpu.make_async_copy(rows.at[b], acc.at[didx.at[j]], ssems[b]).wait()

    gather(0, 0)
    gather(1, 1)
    for j in (0, 1):
        gather_wait(j, j)
        gather(j + 2, j + 2)
        scatter(j, j)

    def body(jj, carry):
        for b4 in range(4):
            j = 2 + 4 * jj + b4
            b = (2 + b4) % 4
            bn = b4
            gather_wait(j, b)
            scatter_wait(j - 2, bn)
            jn = jnp.minimum(j + 2, K16 - 1)
            gather(jn, bn)
            scatter(j, b)
        return carry

    lax.fori_loop(0, (K16 - 2) // 4, body, 0)
    gather_wait(K16 - 1, 2)
    gather_wait(K16 - 1, 3)
    scatter_wait(K16 - 2, 0)
    scatter_wait(K16 - 1, 1)
    plsc.subcore_barrier()
    pltpu.sync_copy(acc.at[pl.ds(s * RPT, RPT)],
                    out_hbm.at[c, pl.ds(s * RPT, RPT)])


def _sc_agg(g, src16, dst16, zeros2d):
    return pl.kernel(
        _agg_body,
        out_type=jax.ShapeDtypeStruct((NC, N_ACC, DF), jnp.float32),
        mesh=_mesh(),
        scratch_types=[
            pltpu.VMEM((K16, CH), jnp.int32),
            pltpu.VMEM((K16, CH), jnp.int32),
            pltpu.VMEM((4, CH, DF), jnp.float32),
            pltpu.VMEM_SHARED((N_ACC, DF), jnp.float32),
            pltpu.SemaphoreType.DMA,
            pltpu.SemaphoreType.DMA,
            pltpu.SemaphoreType.DMA,
            pltpu.SemaphoreType.DMA,
            pltpu.SemaphoreType.DMA,
            pltpu.SemaphoreType.DMA,
            pltpu.SemaphoreType.DMA,
            pltpu.SemaphoreType.DMA,
        ],
        compiler_params=pltpu.CompilerParams(use_tc_tiling_on_sc=False),
    )(g, src16, dst16, zeros2d)


def _mm_scale_body(x_ref, wa_ref, wb_ref, dv_ref, o_ref):
    dv = dv_ref[...]
    x = x_ref[...]
    o_ref[0] = dv * jnp.dot(x, wa_ref[...], preferred_element_type=jnp.float32)
    o_ref[1] = dv * jnp.dot(x, wb_ref[...], preferred_element_type=jnp.float32)


def _tc_mm_scale(x, wa, wb, dinvc):
    return pl.pallas_call(
        _mm_scale_body,
        grid=(GRID,),
        in_specs=[
            pl.BlockSpec((RB, D_IN), lambda i: (i, 0)),
            pl.BlockSpec((D_IN, DF), lambda i: (0, 0)),
            pl.BlockSpec((D_IN, DF), lambda i: (0, 0)),
            pl.BlockSpec((RB, 1), lambda i: (i, 0)),
        ],
        out_specs=pl.BlockSpec((2, RB, DF), lambda i: (0, i, 0)),
        out_shape=jax.ShapeDtypeStruct((2, N, DF), jnp.float32),
    )(x, wa, wb, dinvc)


def _layer2_body(s_ref, g_ref, dv_ref, b_ref, wa_ref, wb_ref, o_ref):
    dv = dv_ref[...]
    ha = dv * (s_ref[0] + g_ref[0]) + b_ref[0]
    hb = dv * (s_ref[1] + g_ref[1]) + b_ref[1]
    pa = (jnp.dot(ha, wa_ref[:DF], preferred_element_type=jnp.float32)
          + jnp.dot(hb, wa_ref[DF:], preferred_element_type=jnp.float32))
    pb = (jnp.dot(ha, wb_ref[:DF], preferred_element_type=jnp.float32)
          + jnp.dot(hb, wb_ref[DF:], preferred_element_type=jnp.float32))
    o_ref[0] = dv * pa
    o_ref[1] = dv * pb


def _tc_layer2(S1, g1, dinvc, b1s, wa, wb):
    return pl.pallas_call(
        _layer2_body,
        grid=(GRID,),
        in_specs=[
            pl.BlockSpec((2, RB, DF), lambda i: (0, i, 0)),
            pl.BlockSpec((2, RB, DF), lambda i: (0, i, 0)),
            pl.BlockSpec((RB, 1), lambda i: (i, 0)),
            pl.BlockSpec((2, DF), lambda i: (0, 0)),
            pl.BlockSpec((D_HID, DF), lambda i: (0, 0)),
            pl.BlockSpec((D_HID, DF), lambda i: (0, 0)),
        ],
        out_specs=pl.BlockSpec((2, RB, DF), lambda i: (0, i, 0)),
        out_shape=jax.ShapeDtypeStruct((2, N, DF), jnp.float32),
    )(S1, g1, dinvc, b1s, wa, wb)


def _final_body(s_ref, g_ref, dv_ref, b_ref, eps_ref, z_ref, mu_ref, ls_ref):
    dv = dv_ref[...]
    mu = dv * (s_ref[0] + g_ref[0]) + b_ref[0]
    ls = dv * (s_ref[1] + g_ref[1]) + b_ref[1]
    mu_ref[...] = mu
    ls_ref[...] = ls
    z_ref[...] = mu + eps_ref[...] * jnp.exp(ls)


def _tc_final(S2, g2, dinvc, bcats, eps):
    return pl.pallas_call(
        _final_body,
        grid=(GRID,),
        in_specs=[
            pl.BlockSpec((2, RB, DF), lambda i: (0, i, 0)),
            pl.BlockSpec((2, RB, DF), lambda i: (0, i, 0)),
            pl.BlockSpec((RB, 1), lambda i: (i, 0)),
            pl.BlockSpec((2, DF), lambda i: (0, 0)),
            pl.BlockSpec((RB, DF), lambda i: (i, 0)),
        ],
        out_specs=[
            pl.BlockSpec((RB, DF), lambda i: (i, 0)),
            pl.BlockSpec((RB, DF), lambda i: (i, 0)),
            pl.BlockSpec((RB, DF), lambda i: (i, 0)),
        ],
        out_shape=[
            jax.ShapeDtypeStruct((N, DF), jnp.float32),
            jax.ShapeDtypeStruct((N, DF), jnp.float32),
            jax.ShapeDtypeStruct((N, DF), jnp.float32),
        ],
    )(S2, g2, dinvc, bcats, eps)


def kernel(x, edge_index, W1, b1, Wmu, bmu, Wls, bls):
    src = edge_index[0]
    dst = edge_index[1]
    pad = E_PAD - E
    ar = jnp.arange(pad, dtype=jnp.int32)
    pad_src = (ar * 37) % N
    pad_dst = N + (ar % PAD_ROWS)
    src_flat = jnp.concatenate([src, pad_src])
    dst_flat = jnp.concatenate([dst, pad_dst])
    dstp32 = dst_flat.reshape(NW, K32, CH)
    src16 = src_flat.reshape(NS, K16, CH)
    dst16 = dst_flat.reshape(NS, K16, CH)

    zeros_n = jnp.zeros((PN,), jnp.float32)
    zeros2d = jnp.zeros((N_ACC, DF), jnp.float32)

    deg_parts = _sc_deg(dstp32, zeros_n)
    deg = deg_parts[0, :N] + deg_parts[1, :N] + 1.0
    dinvc = lax.rsqrt(deg)[:, None]

    g1 = _tc_mm_scale(x, W1[:, :DF], W1[:, DF:], dinvc)
    S1 = _sc_agg(g1, src16, dst16, zeros2d)

    wcat = jnp.concatenate([Wmu, Wls], axis=1)
    g2 = _tc_layer2(S1, g1, dinvc, b1.reshape(2, DF),
                    wcat[:, :DF], wcat[:, DF:])
    S2 = _sc_agg(g2, src16, dst16, zeros2d)

    e = _eps_values()
    eps = (jnp.asarray(e) if e is not None else
           jax.random.normal(jax.random.key(42), (N, D_OUT), jnp.float32))
    z, mu, logstd = _tc_final(S2, g2, dinvc, jnp.stack([bmu, bls]), eps)
    return (z, mu, logstd)


# speedup vs baseline: 1.1151x; 1.1151x over previous
import jax
import jax.numpy as jnp
import numpy as np
from jax import lax
from jax.experimental import pallas as pl
from jax.experimental.pallas import tpu as pltpu
from jax.experimental.pallas import tpu_sc as plsc

N = 10000
E = 320000
D_IN = 128
D_OUT = 64
D_HID = 128
DF = 64

NC = 2
NS = 16
NW = NC * NS
CH = 128
K32 = 81
K16 = 2 * K32
E_PAD = NW * K32 * CH

PAD_ROWS = 112
N_ACC = N + PAD_ROWS
RPT = N_ACC // NS
PN = 10240
DPT = PN // NS

RB = 2000
GRID = N // RB


def _mesh():
    return plsc.VectorSubcoreMesh(core_axis_name="c", subcore_axis_name="s")


_eps_cache = None


def _eps_values():
    global _eps_cache
    if _eps_cache is None:
        try:
            with jax.default_device(jax.devices("cpu")[0]):
                _eps_cache = np.asarray(
                    jax.random.normal(jax.random.key(42), (N, D_OUT),
                                      jnp.float32))
        except Exception:
            return None
    return _eps_cache


def _deg_body(dst_hbm, zeros_hbm, out_hbm, didx, ones_v, acc,
              sem0, sem1, sem2):
    c = lax.axis_index("c")
    s = lax.axis_index("s")
    wid = s * NC + c
    sems = (sem0, sem1, sem2)
    pltpu.sync_copy(dst_hbm.at[wid], didx)
    pltpu.sync_copy(zeros_hbm.at[pl.ds(s * DPT, DPT)], acc.at[pl.ds(s * DPT, DPT)])
    for i in range(CH // 16):
        ones_v[pl.ds(i * 16, 16)] = jnp.ones((16,), jnp.float32)
    plsc.subcore_barrier()

    for b in range(3):
        pltpu.async_copy(ones_v, acc.at[didx.at[b]], sems[b], add=True)

    def body(jj, carry):
        for b in range(3):
            j = 3 * jj + b
            pltpu.make_async_copy(ones_v, acc.at[didx.at[j]], sems[b]).wait()
            pltpu.async_copy(ones_v, acc.at[didx.at[j + 3]], sems[b], add=True)
        return carry

    lax.fori_loop(0, K32 // 3 - 1, body, 0)
    for b in range(3):
        pltpu.make_async_copy(ones_v, acc.at[didx.at[b]], sems[b]).wait()
    plsc.subcore_barrier()
    pltpu.sync_copy(acc.at[pl.ds(s * DPT, DPT)], out_hbm.at[c, pl.ds(s * DPT, DPT)])


def _sc_deg(dstp, zeros_n):
    return pl.kernel(
        _deg_body,
        out_type=jax.ShapeDtypeStruct((NC, PN), jnp.float32),
        mesh=_mesh(),
        scratch_types=[
            pltpu.VMEM((K32, CH), jnp.int32),
            pltpu.VMEM((CH,), jnp.float32),
            pltpu.VMEM_SHARED((PN,), jnp.float32),
            pltpu.SemaphoreType.DMA,
            pltpu.SemaphoreType.DMA,
            pltpu.SemaphoreType.DMA,
        ],
    )(dstp, zeros_n)


def _agg_body(g_hbm, src_hbm, dst_hbm, zeros_hbm, out_hbm, sidx, didx, rows,
              acc, gsem0, gsem1, gsem2):
    c = lax.axis_index("c")
    s = lax.axis_index("s")
    gsems = (gsem0, gsem1, gsem2)
    pltpu.sync_copy(src_hbm.at[s], sidx)
    pltpu.sync_copy(dst_hbm.at[s], didx)
    pltpu.sync_copy(zeros_hbm.at[pl.ds(s * RPT, RPT)], acc.at[pl.ds(s * RPT, RPT)])
    plsc.subcore_barrier()

    gc = g_hbm.at[c]
    for b in range(2):
        pltpu.async_copy(gc.at[sidx.at[b]], rows.at[b], gsems[b])

    def body(jj, carry):
        for b in range(3):
            j = 3 * jj + b
            bn = (b + 2) % 3
            pltpu.make_async_copy(gc.at[sidx.at[j]], rows.at[b], gsems[b]).wait()
            jn = jnp.minimum(j + 2, K16 - 1)
            pltpu.async_copy(gc.at[sidx.at[jn]], rows.at[bn], gsems[bn])
            pltpu.sync_copy(rows.at[b], acc.at[didx.at[j]], add=True)
        return carry

    lax.fori_loop(0, K16 // 3, body, 0)
    for b in range(2):
        pltpu.make_async_copy(gc.at[sidx.at[0]], rows.at[b], gsems[b]).wait()
    plsc.subcore_barrier()
    pltpu.sync_copy(acc.at[pl.ds(s * RPT, RPT)],
                    out_hbm.at[c, pl.ds(s * RPT, RPT)])


def _sc_agg(g, src16, dst16, zeros2d):
    return pl.kernel(
        _agg_body,
        out_type=jax.ShapeDtypeStruct((NC, N_ACC, DF), jnp.float32),
        mesh=_mesh(),
        scratch_types=[
            pltpu.VMEM((K16, CH), jnp.int32),
            pltpu.VMEM((K16, CH), jnp.int32),
            pltpu.VMEM((3, CH, DF), jnp.float32),
            pltpu.VMEM_SHARED((N_ACC, DF), jnp.float32),
            pltpu.SemaphoreType.DMA,
            pltpu.SemaphoreType.DMA,
            pltpu.SemaphoreType.DMA,
        ],
        compiler_params=pltpu.CompilerParams(use_tc_tiling_on_sc=False),
    )(g, src16, dst16, zeros2d)


def _mm_scale_body(x_ref, wa_ref, wb_ref, dv_ref, o_ref):
    dv = dv_ref[...]
    x = x_ref[...]
    o_ref[0] = dv * jnp.dot(x, wa_ref[...], preferred_element_type=jnp.float32)
    o_ref[1] = dv * jnp.dot(x, wb_ref[...], preferred_element_type=jnp.float32)


def _tc_mm_scale(x, wa, wb, dinvc):
    return pl.pallas_call(
        _mm_scale_body,
        grid=(GRID,),
        in_specs=[
            pl.BlockSpec((RB, D_IN), lambda i: (i, 0)),
            pl.BlockSpec((D_IN, DF), lambda i: (0, 0)),
            pl.BlockSpec((D_IN, DF), lambda i: (0, 0)),
            pl.BlockSpec((RB, 1), lambda i: (i, 0)),
        ],
        out_specs=pl.BlockSpec((2, RB, DF), lambda i: (0, i, 0)),
        out_shape=jax.ShapeDtypeStruct((2, N, DF), jnp.float32),
    )(x, wa, wb, dinvc)


def _layer2_body(s_ref, g_ref, dv_ref, b_ref, wa_ref, wb_ref, o_ref):
    dv = dv_ref[...]
    ha = dv * (s_ref[0] + g_ref[0]) + b_ref[0]
    hb = dv * (s_ref[1] + g_ref[1]) + b_ref[1]
    pa = (jnp.dot(ha, wa_ref[:DF], preferred_element_type=jnp.float32)
          + jnp.dot(hb, wa_ref[DF:], preferred_element_type=jnp.float32))
    pb = (jnp.dot(ha, wb_ref[:DF], preferred_element_type=jnp.float32)
          + jnp.dot(hb, wb_ref[DF:], preferred_element_type=jnp.float32))
    o_ref[0] = dv * pa
    o_ref[1] = dv * pb


def _tc_layer2(S1, g1, dinvc, b1s, wa, wb):
    return pl.pallas_call(
        _layer2_body,
        grid=(GRID,),
        in_specs=[
            pl.BlockSpec((2, RB, DF), lambda i: (0, i, 0)),
            pl.BlockSpec((2, RB, DF), lambda i: (0, i, 0)),
            pl.BlockSpec((RB, 1), lambda i: (i, 0)),
            pl.BlockSpec((2, DF), lambda i: (0, 0)),
            pl.BlockSpec((D_HID, DF), lambda i: (0, 0)),
            pl.BlockSpec((D_HID, DF), lambda i: (0, 0)),
        ],
        out_specs=pl.BlockSpec((2, RB, DF), lambda i: (0, i, 0)),
        out_shape=jax.ShapeDtypeStruct((2, N, DF), jnp.float32),
    )(S1, g1, dinvc, b1s, wa, wb)


def _final_body(s_ref, g_ref, dv_ref, b_ref, eps_ref, z_ref, mu_ref, ls_ref):
    dv = dv_ref[...]
    mu = dv * (s_ref[0] + g_ref[0]) + b_ref[0]
    ls = dv * (s_ref[1] + g_ref[1]) + b_ref[1]
    mu_ref[...] = mu
    ls_ref[...] = ls
    z_ref[...] = mu + eps_ref[...] * jnp.exp(ls)


def _tc_final(S2, g2, dinvc, bcats, eps):
    return pl.pallas_call(
        _final_body,
        grid=(GRID,),
        in_specs=[
            pl.BlockSpec((2, RB, DF), lambda i: (0, i, 0)),
            pl.BlockSpec((2, RB, DF), lambda i: (0, i, 0)),
            pl.BlockSpec((RB, 1), lambda i: (i, 0)),
            pl.BlockSpec((2, DF), lambda i: (0, 0)),
            pl.BlockSpec((RB, DF), lambda i: (i, 0)),
        ],
        out_specs=[
            pl.BlockSpec((RB, DF), lambda i: (i, 0)),
            pl.BlockSpec((RB, DF), lambda i: (i, 0)),
            pl.BlockSpec((RB, DF), lambda i: (i, 0)),
        ],
        out_shape=[
            jax.ShapeDtypeStruct((N, DF), jnp.float32),
            jax.ShapeDtypeStruct((N, DF), jnp.float32),
            jax.ShapeDtypeStruct((N, DF), jnp.float32),
        ],
    )(S2, g2, dinvc, bcats, eps)


def kernel(x, edge_index, W1, b1, Wmu, bmu, Wls, bls):
    src = edge_index[0]
    dst = edge_index[1]
    pad = E_PAD - E
    ar = jnp.arange(pad, dtype=jnp.int32)
    pad_src = (ar * 37) % N
    pad_dst = N + (ar % PAD_ROWS)
    src_flat = jnp.concatenate([src, pad_src])
    dst_flat = jnp.concatenate([dst, pad_dst])
    dstp32 = dst_flat.reshape(NW, K32, CH)
    src16 = src_flat.reshape(NS, K16, CH)
    dst16 = dst_flat.reshape(NS, K16, CH)

    zeros_n = jnp.zeros((PN,), jnp.float32)
    zeros2d = jnp.zeros((N_ACC, DF), jnp.float32)

    deg_parts = _sc_deg(dstp32, zeros_n)
    deg = deg_parts[0, :N] + deg_parts[1, :N] + 1.0
    dinvc = lax.rsqrt(deg)[:, None]

    g1 = _tc_mm_scale(x, W1[:, :DF], W1[:, DF:], dinvc)
    S1 = _sc_agg(g1, src16, dst16, zeros2d)

    wcat = jnp.concatenate([Wmu, Wls], axis=1)
    g2 = _tc_layer2(S1, g1, dinvc, b1.reshape(2, DF),
                    wcat[:, :DF], wcat[:, DF:])
    S2 = _sc_agg(g2, src16, dst16, zeros2d)

    e = _eps_values()
    eps = (jnp.asarray(e) if e is not None else
           jax.random.normal(jax.random.key(42), (N, D_OUT), jnp.float32))
    z, mu, logstd = _tc_final(S2, g2, dinvc, jnp.stack([bmu, bls]), eps)
    return (z, mu, logstd)
